# baseline (speedup 1.0000x reference)
import functools

import jax
import jax.numpy as jnp
from jax import lax
from jax.experimental import pallas as pl
from jax.experimental.pallas import tpu as pltpu
from jax.experimental.pallas import tpu_sc as plsc

E = 8
K = 2
D = 1024
DFF = 2048
N = 2048
A = K * N
T = 128
G = A + E * T
NTILES = G // T

NC = 2
NS = 16
NW = NC * NS

@functools.cache
def _sc_mesh():
    return plsc.VectorSubcoreMesh(
        core_axis_name="c", subcore_axis_name="s", num_cores=NC, num_subcores=NS
    )


def _gating_body(x_ref, wg_ref, loss_ref, pos1_ref, pos2_ref, p1t_ref, p2t_ref,
                 g1t_ref, g2t_ref, te_ref):
    logits = jnp.dot(x_ref[...], wg_ref[...], preferred_element_type=jnp.float32)
    m = jnp.max(logits, axis=1, keepdims=True)
    ex = jnp.exp(logits - m)
    probs = ex / jnp.sum(ex, axis=1, keepdims=True)

    cols = lax.broadcasted_iota(jnp.int32, (N, E), 1)
    max1 = jnp.max(probs, axis=1, keepdims=True)
    idx1 = jnp.min(jnp.where(probs == max1, cols, E), axis=1, keepdims=True)
    m1 = cols == idx1
    pm = jnp.where(m1, -1.0, probs)
    max2 = jnp.max(pm, axis=1, keepdims=True)
    idx2 = jnp.min(jnp.where(pm == max2, cols, E), axis=1, keepdims=True)
    m2 = cols == idx2

    denom = max1 + max2
    g1 = max1 / denom
    g2 = max2 / denom
    g1t_ref[...] = g1.reshape(1, N)
    g2t_ref[...] = g2.reshape(1, N)
    gates = jnp.where(m1, g1, 0.0) + jnp.where(m2, g2, 0.0)

    importance = jnp.sum(gates, axis=0)
    load = jnp.sum((gates > 0.0).astype(jnp.float32), axis=0)

    def cv_sq(v):
        mean = jnp.sum(v) / E
        var = jnp.sum((v - mean) ** 2) / (E - 1)
        return var / (mean * mean + 1e-10)

    loss_ref[...] = jnp.broadcast_to(
        0.01 * (cv_sq(importance) + cv_sq(load)), (1, 1)
    )

    sel = (m1 | m2).astype(jnp.float32)
    incl = sel
    sh = 1
    while sh < N:
        shifted = jnp.concatenate(
            [jnp.zeros((sh, E), jnp.float32), incl[: N - sh, :]], axis=0
        )
        incl = incl + shifted
        sh *= 2
    rank = incl - sel

    counts = jnp.sum(sel, axis=0, keepdims=True)
    pc = jnp.floor((counts + (T - 1)) * (1.0 / T)) * T
    r8 = lax.broadcasted_iota(jnp.int32, (E, E), 0)
    c8 = lax.broadcasted_iota(jnp.int32, (E, E), 1)
    upper_incl = (r8 <= c8).astype(jnp.float32)
    ends = jnp.dot(pc, upper_incl, preferred_element_type=jnp.float32)
    base = ends - pc

    posd = base + rank
    pos1 = jnp.sum(jnp.where(m1, posd, 0.0), axis=1, keepdims=True)
    pos2 = jnp.sum(jnp.where(m2, posd, 0.0), axis=1, keepdims=True)
    pos1i = pos1.astype(jnp.int32)
    pos2i = pos2.astype(jnp.int32)
    pos1_ref[...] = pos1i
    pos2_ref[...] = pos2i
    p1t_ref[...] = pos1i.reshape(1, N)
    p2t_ref[...] = pos2i.reshape(1, N)

    jT = lax.broadcasted_iota(jnp.int32, (NTILES, E), 0).astype(jnp.float32) * T
    endsb = jnp.broadcast_to(ends, (NTILES, E))
    te = jnp.sum((jT >= endsb).astype(jnp.float32), axis=1, keepdims=True)
    total = jnp.max(ends)
    valid = jT[:, :1] < total
    te_ref[...] = jnp.where(valid, te, -1.0).astype(jnp.int32)


def _gating(xf, w_gate):
    return pl.pallas_call(
        _gating_body,
        out_shape=(
            jax.ShapeDtypeStruct((1, 1), jnp.float32),
            jax.ShapeDtypeStruct((N, 1), jnp.int32),
            jax.ShapeDtypeStruct((N, 1), jnp.int32),
            jax.ShapeDtypeStruct((1, N), jnp.int32),
            jax.ShapeDtypeStruct((1, N), jnp.int32),
            jax.ShapeDtypeStruct((1, N), jnp.float32),
            jax.ShapeDtypeStruct((1, N), jnp.float32),
            jax.ShapeDtypeStruct((NTILES, 1), jnp.int32),
        ),
    )(xf, w_gate)


def _mm_body(te_ref, x_ref, p1t_ref, p2t_ref, g1t_ref, g2t_ref, w1_ref, b1_ref,
             w2_ref, b2_ref, og_ref):
    j = pl.program_id(0)

    @pl.when(te_ref[j] >= 0)
    def _():
        rowi = lax.broadcasted_iota(jnp.int32, (T, N), 0) + j * T
        m1 = p1t_ref[...] == rowi
        m2 = p2t_ref[...] == rowi
        sel = (m1 | m2).astype(jnp.bfloat16)
        xt = jnp.dot(sel, x_ref[...], preferred_element_type=jnp.float32)
        gcol = jnp.sum(
            jnp.where(m1, g1t_ref[...], 0.0) + jnp.where(m2, g2t_ref[...], 0.0),
            axis=1,
            keepdims=True,
        )
        h = jnp.maximum(
            jnp.dot(xt, w1_ref[0], preferred_element_type=jnp.float32)
            + b1_ref[0],
            0.0,
        )
        o = jnp.dot(h, w2_ref[0], preferred_element_type=jnp.float32) + b2_ref[0]
        og_ref[...] = gcol * o


def _grouped_mm(te, xf, p1t, p2t, g1t, g2t, w1, b1, w2, b2):
    def eidx(j, te_ref):
        return jnp.where(te_ref[j] < 0, E - 1, te_ref[j])

    return pl.pallas_call(
        _mm_body,
        grid_spec=pltpu.PrefetchScalarGridSpec(
            num_scalar_prefetch=1,
            grid=(NTILES,),
            in_specs=[
                pl.BlockSpec((N, D), lambda j, te_ref: (0, 0)),
                pl.BlockSpec((1, N), lambda j, te_ref: (0, 0)),
                pl.BlockSpec((1, N), lambda j, te_ref: (0, 0)),
                pl.BlockSpec((1, N), lambda j, te_ref: (0, 0)),
                pl.BlockSpec((1, N), lambda j, te_ref: (0, 0)),
                pl.BlockSpec((1, D, DFF), lambda j, te_ref: (eidx(j, te_ref), 0, 0)),
                pl.BlockSpec((1, 1, DFF), lambda j, te_ref: (eidx(j, te_ref), 0, 0)),
                pl.BlockSpec((1, DFF, D), lambda j, te_ref: (eidx(j, te_ref), 0, 0)),
                pl.BlockSpec((1, 1, D), lambda j, te_ref: (eidx(j, te_ref), 0, 0)),
            ],
            out_specs=pl.BlockSpec((T, D), lambda j, te_ref: (j, 0)),
        ),
        out_shape=jax.ShapeDtypeStruct((G, D), jnp.float32),
    )(te, xf.astype(jnp.bfloat16), p1t, p2t, g1t, g2t, w1,
      b1.reshape(E, 1, DFF), w2, b2.reshape(E, 1, D))


_CROWS = N // NW
_CCHUNK = 16
_CNCH = _CROWS // _CCHUNK

@functools.cache
def _sc_combine_kernel():
    return pl.kernel(
        _sc_combine_body,
        out_type=jax.ShapeDtypeStruct((N, D), jnp.float32),
        mesh=_sc_mesh(),
        scratch_types=(
            pltpu.VMEM((_CROWS,), jnp.int32),
            pltpu.VMEM((_CROWS,), jnp.int32),
            pltpu.VMEM((_CCHUNK, D), jnp.float32),
            pltpu.VMEM((_CCHUNK, D), jnp.float32),
            pltpu.VMEM((_CCHUNK, D), jnp.float32),
            pltpu.VMEM((_CCHUNK, D), jnp.float32),
            pltpu.SemaphoreType.DMA,
            pltpu.SemaphoreType.DMA,
            pltpu.SemaphoreType.DMA,
            pltpu.SemaphoreType.DMA,
            pltpu.SemaphoreType.DMA,
            pltpu.SemaphoreType.DMA,
        ),
    )


def _sc_combine_body(pos1_hbm, pos2_hbm, og_hbm, y_hbm, pa_v, pb_v, a0, a1,
                     b0, b1, sa0, sa1, sb0, sb1, sw0, sw1):
    wid = lax.axis_index("s") * NC + lax.axis_index("c")
    base = wid * _CROWS
    pltpu.sync_copy(pos1_hbm.at[pl.ds(base, _CROWS)], pa_v)
    pltpu.sync_copy(pos2_hbm.at[pl.ds(base, _CROWS)], pb_v)
    av = (a0, a1)
    bv = (b0, b1)
    sa = (sa0, sa1)
    sb = (sb0, sb1)
    sw = (sw0, sw1)
    gha = [None, None]
    ghb = [None, None]
    wh = [None, None]

    def start_gather(j):
        k = j & 1
        sl = pl.ds(j * _CCHUNK, _CCHUNK)
        gha[k] = pltpu.async_copy(og_hbm.at[pa_v.at[sl]], av[k], sa[k])
        ghb[k] = pltpu.async_copy(og_hbm.at[pb_v.at[sl]], bv[k], sb[k])

    start_gather(0)
    for j in range(_CNCH):
        k = j & 1
        if j + 1 < _CNCH:
            if wh[1 - k] is not None:
                wh[1 - k].wait()
                wh[1 - k] = None
            start_gather(j + 1)
        gha[k].wait()
        ghb[k].wait()
        a_v, b_v = av[k], bv[k]

        def add_body(i, _):
            r = i >> 3
            cb = (i & 7) * 128
            for v in range(8):
                sl = pl.ds(cb + v * 16, 16)
                a_v[r, sl] = a_v[r, sl] + b_v[r, sl]
            return 0

        lax.fori_loop(0, _CCHUNK * 8, add_body, 0)
        wh[k] = pltpu.async_copy(
            a_v, y_hbm.at[pl.ds(base + j * _CCHUNK, _CCHUNK)], sw[k]
        )
    for k in range(2):
        if wh[k] is not None:
            wh[k].wait()


def kernel(x, w_gate, w1, b1, w2, b2):
    xf = x.reshape(-1, D)

    loss, pos1, pos2, p1t, p2t, g1t, g2t, te = _gating(xf, w_gate)

    og = _grouped_mm(te.reshape(NTILES), xf, p1t, p2t, g1t, g2t,
                     w1, b1, w2, b2)
    y = _sc_combine_kernel()(pos1.reshape(N), pos2.reshape(N), og)

    return (y.reshape(x.shape), loss.reshape(()))


# speedup vs baseline: 1.0333x; 1.0333x over previous
import functools

import jax
import jax.numpy as jnp
from jax import lax
from jax.experimental import pallas as pl
from jax.experimental.pallas import tpu as pltpu
from jax.experimental.pallas import tpu_sc as plsc

E = 8
K = 2
D = 1024
DFF = 2048
N = 2048
A = K * N
T = 128
G = A + E * T
NTILES = G // T

NC = 2
NS = 16
NW = NC * NS

@functools.cache
def _sc_mesh():
    return plsc.VectorSubcoreMesh(
        core_axis_name="c", subcore_axis_name="s", num_cores=NC, num_subcores=NS
    )


def _gating_body(x_ref, wg_ref, loss_ref, pos1_ref, pos2_ref, p1t_ref, p2t_ref,
                 g1t_ref, g2t_ref, te_ref):
    logits = jnp.dot(x_ref[...], wg_ref[...], preferred_element_type=jnp.float32)
    m = jnp.max(logits, axis=1, keepdims=True)
    ex = jnp.exp(logits - m)
    probs = ex / jnp.sum(ex, axis=1, keepdims=True)

    cols = lax.broadcasted_iota(jnp.int32, (N, E), 1)
    max1 = jnp.max(probs, axis=1, keepdims=True)
    idx1 = jnp.min(jnp.where(probs == max1, cols, E), axis=1, keepdims=True)
    m1 = cols == idx1
    pm = jnp.where(m1, -1.0, probs)
    max2 = jnp.max(pm, axis=1, keepdims=True)
    idx2 = jnp.min(jnp.where(pm == max2, cols, E), axis=1, keepdims=True)
    m2 = cols == idx2

    denom = max1 + max2
    g1 = max1 / denom
    g2 = max2 / denom
    g1t_ref[...] = g1.reshape(1, N)
    g2t_ref[...] = g2.reshape(1, N)
    gates = jnp.where(m1, g1, 0.0) + jnp.where(m2, g2, 0.0)

    importance = jnp.sum(gates, axis=0)
    load = jnp.sum((gates > 0.0).astype(jnp.float32), axis=0)

    def cv_sq(v):
        mean = jnp.sum(v) / E
        var = jnp.sum((v - mean) ** 2) / (E - 1)
        return var / (mean * mean + 1e-10)

    loss_ref[...] = jnp.broadcast_to(
        0.01 * (cv_sq(importance) + cv_sq(load)), (1, 1)
    )

    sel = (m1 | m2).astype(jnp.float32)
    incl = sel
    sh = 1
    while sh < N:
        shifted = jnp.concatenate(
            [jnp.zeros((sh, E), jnp.float32), incl[: N - sh, :]], axis=0
        )
        incl = incl + shifted
        sh *= 2
    rank = incl - sel

    counts = jnp.sum(sel, axis=0, keepdims=True)
    pc = jnp.floor((counts + (T - 1)) * (1.0 / T)) * T
    r8 = lax.broadcasted_iota(jnp.int32, (E, E), 0)
    c8 = lax.broadcasted_iota(jnp.int32, (E, E), 1)
    upper_incl = (r8 <= c8).astype(jnp.float32)
    ends = jnp.dot(pc, upper_incl, preferred_element_type=jnp.float32)
    base = ends - pc

    posd = base + rank
    pos1 = jnp.sum(jnp.where(m1, posd, 0.0), axis=1, keepdims=True)
    pos2 = jnp.sum(jnp.where(m2, posd, 0.0), axis=1, keepdims=True)
    pos1i = pos1.astype(jnp.int32)
    pos2i = pos2.astype(jnp.int32)
    pos1_ref[...] = pos1i
    pos2_ref[...] = pos2i
    p1t_ref[...] = pos1i.reshape(1, N)
    p2t_ref[...] = pos2i.reshape(1, N)

    jT = lax.broadcasted_iota(jnp.int32, (NTILES, E), 0).astype(jnp.float32) * T
    endsb = jnp.broadcast_to(ends, (NTILES, E))
    te = jnp.sum((jT >= endsb).astype(jnp.float32), axis=1, keepdims=True)
    total = jnp.max(ends)
    valid = jT[:, :1] < total
    te_ref[...] = jnp.where(valid, te, -1.0).astype(jnp.int32)


def _gating(xf, w_gate):
    return pl.pallas_call(
        _gating_body,
        out_shape=(
            jax.ShapeDtypeStruct((1, 1), jnp.float32),
            jax.ShapeDtypeStruct((N, 1), jnp.int32),
            jax.ShapeDtypeStruct((N, 1), jnp.int32),
            jax.ShapeDtypeStruct((1, N), jnp.int32),
            jax.ShapeDtypeStruct((1, N), jnp.int32),
            jax.ShapeDtypeStruct((1, N), jnp.float32),
            jax.ShapeDtypeStruct((1, N), jnp.float32),
            jax.ShapeDtypeStruct((NTILES, 1), jnp.int32),
        ),
    )(xf, w_gate)


def _mm_body(te_ref, x_ref, p1t_ref, p2t_ref, g1t_ref, g2t_ref, w1_ref, b1_ref,
             w2_ref, b2_ref, og_ref):
    j = pl.program_id(0)

    @pl.when(te_ref[j] >= 0)
    def _():
        rowi = lax.broadcasted_iota(jnp.int32, (T, N), 0) + j * T
        m1 = p1t_ref[...] == rowi
        m2 = p2t_ref[...] == rowi
        sel = (m1 | m2).astype(jnp.float32)
        xt = jnp.dot(sel, x_ref[...], preferred_element_type=jnp.float32)
        gcol = jnp.sum(
            jnp.where(m1, g1t_ref[...], 0.0) + jnp.where(m2, g2t_ref[...], 0.0),
            axis=1,
            keepdims=True,
        )
        h = jnp.maximum(
            jnp.dot(xt, w1_ref[0], preferred_element_type=jnp.float32)
            + b1_ref[0],
            0.0,
        )
        o = jnp.dot(h, w2_ref[0], preferred_element_type=jnp.float32) + b2_ref[0]
        og_ref[...] = gcol * o


def _grouped_mm(te, xf, p1t, p2t, g1t, g2t, w1, b1, w2, b2):
    def eidx(j, te_ref):
        return jnp.where(te_ref[j] < 0, E - 1, te_ref[j])

    return pl.pallas_call(
        _mm_body,
        grid_spec=pltpu.PrefetchScalarGridSpec(
            num_scalar_prefetch=1,
            grid=(NTILES,),
            in_specs=[
                pl.BlockSpec((N, D), lambda j, te_ref: (0, 0)),
                pl.BlockSpec((1, N), lambda j, te_ref: (0, 0)),
                pl.BlockSpec((1, N), lambda j, te_ref: (0, 0)),
                pl.BlockSpec((1, N), lambda j, te_ref: (0, 0)),
                pl.BlockSpec((1, N), lambda j, te_ref: (0, 0)),
                pl.BlockSpec((1, D, DFF), lambda j, te_ref: (eidx(j, te_ref), 0, 0)),
                pl.BlockSpec((1, 1, DFF), lambda j, te_ref: (eidx(j, te_ref), 0, 0)),
                pl.BlockSpec((1, DFF, D), lambda j, te_ref: (eidx(j, te_ref), 0, 0)),
                pl.BlockSpec((1, 1, D), lambda j, te_ref: (eidx(j, te_ref), 0, 0)),
            ],
            out_specs=pl.BlockSpec((T, D), lambda j, te_ref: (j, 0)),
        ),
        out_shape=jax.ShapeDtypeStruct((G, D), jnp.float32),
    )(te, xf, p1t, p2t, g1t, g2t, w1,
      b1.reshape(E, 1, DFF), w2, b2.reshape(E, 1, D))


_CROWS = N // NW
_CCHUNK = 16
_CNCH = _CROWS // _CCHUNK

@functools.cache
def _sc_combine_kernel():
    return pl.kernel(
        _sc_combine_body,
        out_type=jax.ShapeDtypeStruct((N, D), jnp.float32),
        mesh=_sc_mesh(),
        scratch_types=(
            pltpu.VMEM((_CROWS,), jnp.int32),
            pltpu.VMEM((_CROWS,), jnp.int32),
            pltpu.VMEM((_CCHUNK, D), jnp.float32),
            pltpu.VMEM((_CCHUNK, D), jnp.float32),
            pltpu.VMEM((_CCHUNK, D), jnp.float32),
            pltpu.VMEM((_CCHUNK, D), jnp.float32),
            pltpu.SemaphoreType.DMA,
            pltpu.SemaphoreType.DMA,
            pltpu.SemaphoreType.DMA,
            pltpu.SemaphoreType.DMA,
            pltpu.SemaphoreType.DMA,
            pltpu.SemaphoreType.DMA,
        ),
    )


def _sc_combine_body(pos1_hbm, pos2_hbm, og_hbm, y_hbm, pa_v, pb_v, a0, a1,
                     b0, b1, sa0, sa1, sb0, sb1, sw0, sw1):
    wid = lax.axis_index("s") * NC + lax.axis_index("c")
    base = wid * _CROWS
    pltpu.sync_copy(pos1_hbm.at[pl.ds(base, _CROWS)], pa_v)
    pltpu.sync_copy(pos2_hbm.at[pl.ds(base, _CROWS)], pb_v)
    av = (a0, a1)
    bv = (b0, b1)
    sa = (sa0, sa1)
    sb = (sb0, sb1)
    sw = (sw0, sw1)
    gha = [None, None]
    ghb = [None, None]
    wh = [None, None]

    def start_gather(j):
        k = j & 1
        sl = pl.ds(j * _CCHUNK, _CCHUNK)
        gha[k] = pltpu.async_copy(og_hbm.at[pa_v.at[sl]], av[k], sa[k])
        ghb[k] = pltpu.async_copy(og_hbm.at[pb_v.at[sl]], bv[k], sb[k])

    start_gather(0)
    for j in range(_CNCH):
        k = j & 1
        if j + 1 < _CNCH:
            if wh[1 - k] is not None:
                wh[1 - k].wait()
                wh[1 - k] = None
            start_gather(j + 1)
        gha[k].wait()
        ghb[k].wait()
        a_v, b_v = av[k], bv[k]

        def add_body(i, _):
            r = i >> 3
            cb = (i & 7) * 128
            for v in range(8):
                sl = pl.ds(cb + v * 16, 16)
                a_v[r, sl] = a_v[r, sl] + b_v[r, sl]
            return 0

        lax.fori_loop(0, _CCHUNK * 8, add_body, 0)
        wh[k] = pltpu.async_copy(
            a_v, y_hbm.at[pl.ds(base + j * _CCHUNK, _CCHUNK)], sw[k]
        )
    for k in range(2):
        if wh[k] is not None:
            wh[k].wait()


def kernel(x, w_gate, w1, b1, w2, b2):
    xf = x.reshape(-1, D)

    loss, pos1, pos2, p1t, p2t, g1t, g2t, te = _gating(xf, w_gate)

    og = _grouped_mm(te.reshape(NTILES), xf, p1t, p2t, g1t, g2t,
                     w1, b1, w2, b2)
    y = _sc_combine_kernel()(pos1.reshape(N), pos2.reshape(N), og)

    return (y.reshape(x.shape), loss.reshape(()))
